# Initial kernel scaffold; baseline (speedup 1.0000x reference)
import jax
import jax.numpy as jnp
from jax.experimental import pallas as pl


def kernel(embs, indices):
    raise NotImplementedError("write your pallas kernel here")


# speedup vs baseline: 1.5355x; 1.5355x over previous
import jax
import jax.numpy as jnp
from jax.experimental import pallas as pl
from jax.experimental.pallas import tpu as pltpu

_B = 128
_MARGIN = 1.0
_EPS = 1e-8


def _tc_body(embs_ref, lab_col_ref, lab_row_ref, out_ref, apt_ref, ant_ref):
    e = embs_ref[...]
    g = jax.lax.dot_general(e, e, (((1,), (1,)), ((), ())),
                            preferred_element_type=jnp.float32)
    n2c = jnp.sum(e * e, axis=1, keepdims=True)
    riota = jax.lax.broadcasted_iota(jnp.int32, (_B, _B), 0)
    ciota = jax.lax.broadcasted_iota(jnp.int32, (_B, _B), 1)
    eye = riota == ciota
    diag_n2 = jnp.where(eye, jnp.broadcast_to(n2c, (_B, _B)), 0.0)
    n2r = jax.lax.dot_general(jnp.ones((_B, _B), jnp.float32), diag_n2,
                              (((1,), (0,)), ((), ())),
                              preferred_element_type=jnp.float32)
    denom = jnp.maximum(jnp.sqrt(jnp.broadcast_to(n2c, (_B, _B)) * n2r), _EPS)
    s = g / denom

    lab_c = jnp.broadcast_to(lab_col_ref[...], (_B, _B))
    lab_r = jnp.broadcast_to(lab_row_ref[...], (_B, _B))
    same = lab_c == lab_r
    gt_t = riota > ciota
    apt_ref[...] = jnp.where(gt_t & same, s, 3.0)
    ant_ref[...] = jnp.where(gt_t & (~same), s + _MARGIN, -3.0)

    ant = ant_ref[...]

    def jbody(j, acc):
        prow = apt_ref[pl.ds(j, 1), :]
        return acc + jnp.maximum(ant - prow, 0.0)

    acc = jax.lax.fori_loop(0, _B, jbody, jnp.zeros((_B, _B), jnp.float32))
    out_ref[...] = jnp.sum(acc, keepdims=True)


def kernel(embs, indices):
    lab = indices.astype(jnp.int32)
    out = pl.pallas_call(
        _tc_body,
        out_shape=jax.ShapeDtypeStruct((1, 1), jnp.float32),
        scratch_shapes=[
            pltpu.VMEM((_B, _B), jnp.float32),
            pltpu.VMEM((_B, _B), jnp.float32),
        ],
    )(embs, lab.reshape(_B, 1), lab.reshape(1, _B))
    return out[0, 0]
